# baseline (speedup 1.0000x reference)
import functools

import jax
import jax.numpy as jnp
from jax import lax
from jax.experimental import pallas as pl
from jax.experimental.pallas import tpu as pltpu
from jax.experimental.pallas import tpu_sc as plsc

B = 4
S = 2048
D = 1024
L = 10
E = 16
K = 2
CHUNK = 256
NSTEP = S // CHUNK

_SC_INFO = plsc.get_sparse_core_info()
_NC = _SC_INFO.num_cores
_NS = _SC_INFO.num_subcores
_NW = _NC * _NS
ROWS_OUT = B * K * L


def _chunk_sum(x):
    n = x.shape[0]
    p = [x[0], x[1], x[2], x[3]]
    for g in range(1, n // 4):
        for k in range(4):
            p[k] = p[k] + x[4 * g + k]
    return (p[0] + p[1]) + (p[2] + p[3])


def _tc_body(x_ref, w_ref, b_ref, out_ref, acc_ref):
    i = pl.program_id(0)

    @pl.when(i == 0)
    def _init():
        acc_ref[...] = jnp.zeros_like(acc_ref)

    x = x_ref[...].reshape(B, CHUNK // 8, 8, D)
    for b in range(B):
        acc_ref[b] += _chunk_sum(x[b])

    @pl.when(i == NSTEP - 1)
    def _finish():
        mean = jnp.sum(acc_ref[...], axis=1) * (1.0 / S)
        out_ref[...] = jax.lax.dot_general(
            mean, w_ref[...], (((1,), (1,)), ((), ())),
            preferred_element_type=jnp.float32) + b_ref[...]


def _router_logits(x_embed, router_w, router_b):
    return pl.pallas_call(
        _tc_body,
        grid=(NSTEP,),
        in_specs=[
            pl.BlockSpec((B, CHUNK, D), lambda i: (0, i, 0)),
            pl.BlockSpec((E, D), lambda i: (0, 0)),
            pl.BlockSpec((1, E), lambda i: (0, 0)),
        ],
        out_specs=pl.BlockSpec((B, E), lambda i: (0, 0)),
        out_shape=jax.ShapeDtypeStruct((B, E), jnp.float32),
        scratch_shapes=[pltpu.VMEM((B, 8, D), jnp.float32)],
        compiler_params=pltpu.CompilerParams(
            dimension_semantics=("arbitrary",)),
    )(x_embed, router_w, router_b.reshape(1, E))


def _sc_body(logits_hbm, p_hbm, out_hbm,
             logits_v, scores_tab, experts_tab, idx_ref, rowbuf, sem):
    wid = lax.axis_index("s") * _NC + lax.axis_index("c")
    lane = lax.iota(jnp.int32, 16)

    pltpu.sync_copy(logits_hbm, logits_v)

    def perm(v, idx):
        dn = lax.GatherDimensionNumbers(
            offset_dims=(), collapsed_slice_dims=(0,), start_index_map=(0,))
        return lax.gather(v, idx[:, None], dn, slice_sizes=(1,),
                          mode=lax.GatherScatterMode.PROMISE_IN_BOUNDS)

    def bfly(v, op):
        for sh in (1, 2, 4, 8):
            v = op(v, perm(v, lane ^ sh))
        return v

    for b in range(B):
        lv = logits_v[b]
        e = jnp.exp(lv)
        scores = e / bfly(e, jnp.add)
        m1 = bfly(scores, jnp.maximum)
        i1 = bfly(jnp.where(scores == m1, lane, E), jnp.minimum)
        s2 = jnp.where(lane == i1, -1.0, scores)
        m2 = bfly(s2, jnp.maximum)
        i2 = bfly(jnp.where(s2 == m2, lane, E), jnp.minimum)
        scores_tab[b * K + 0] = m1
        scores_tab[b * K + 1] = m2
        experts_tab[b * K + 0] = i1
        experts_tab[b * K + 1] = i2

    for mstep in range((ROWS_OUT + _NW - 1) // _NW):
        r = wid + _NW * mstep

        @pl.when(r < ROWS_OUT)
        def _row():
            bk = r // L
            l = r - bk * L
            score_vec = scores_tab[bk]
            expert_vec = experts_tab[bk]
            row_id = expert_vec * L + l
            idx_ref[...] = row_id
            pltpu.async_copy(p_hbm.at[idx_ref], rowbuf, sem).wait()
            for c in range(D // 16):
                sl = pl.ds(c * 16, 16)
                rowbuf[0, sl] = rowbuf[0, sl] * score_vec
            pltpu.sync_copy(rowbuf.at[pl.ds(0, 1), :],
                            out_hbm.at[pl.ds(r, 1), :])


@functools.partial(
    pl.kernel,
    out_type=jax.ShapeDtypeStruct((ROWS_OUT, D), jnp.float32),
    mesh=plsc.VectorSubcoreMesh(core_axis_name="c", subcore_axis_name="s"),
    scratch_types=[
        pltpu.VMEM((B, E), jnp.float32),
        pltpu.VMEM((B * K, 16), jnp.float32),
        pltpu.VMEM((B * K, 16), jnp.int32),
        pltpu.VMEM((16,), jnp.int32),
        pltpu.VMEM((16, D), jnp.float32),
        pltpu.SemaphoreType.DMA,
    ],
)
def _sc_mix(logits_hbm, p_hbm, out_hbm,
            logits_v, scores_tab, experts_tab, idx_ref, rowbuf, sem):
    _sc_body(logits_hbm, p_hbm, out_hbm,
             logits_v, scores_tab, experts_tab, idx_ref, rowbuf, sem)


@jax.jit
def _run(x_embed, prompts, router_w, router_b):
    logits = _router_logits(x_embed, router_w, router_b)
    p2d = prompts.reshape(E * L, D)
    out = _sc_mix(logits, p2d)
    return out.reshape(B, K * L, D)


def kernel(x_embed, prompts, router_w, router_b, layer_idx):
    return _run(x_embed, prompts, router_w, router_b)
